# Initial kernel scaffold; baseline (speedup 1.0000x reference)
import jax
import jax.numpy as jnp
from jax.experimental import pallas as pl


def kernel(query, key, value):
    raise NotImplementedError("write your pallas kernel here")


# speedup vs baseline: 1.5030x; 1.5030x over previous
import functools

import jax
import jax.numpy as jnp
from jax.experimental import pallas as pl
from jax.experimental.pallas import tpu as pltpu

B, H, S, D = 1, 16, 2048, 128
BQ = 512


def _attn_block(q_ref, k_ref, v_ref, o_ref, *, scale):
    q = q_ref[0] * scale
    k = k_ref[0]
    s = jax.lax.dot_general(q, k, (((1,), (1,)), ((), ())),
                            preferred_element_type=jnp.float32)
    m = jnp.max(s, axis=1, keepdims=True)
    p = jnp.exp(s - m)
    l = jnp.sum(p, axis=1, keepdims=True)
    o = jax.lax.dot_general(p, v_ref[0], (((1,), (0,)), ((), ())),
                            preferred_element_type=jnp.float32)
    o_ref[0] = o / l


def kernel(query, key, value):
    scale = D ** (-0.5)
    q = query.reshape(H, S, D)
    k = key.reshape(H, S, D)
    v = value.reshape(H, S, D)
    out = pl.pallas_call(
        functools.partial(_attn_block, scale=scale),
        grid=(H, S // BQ),
        in_specs=[
            pl.BlockSpec((1, BQ, D), lambda h, i: (h, i, 0)),
            pl.BlockSpec((1, S, D), lambda h, i: (h, 0, 0)),
            pl.BlockSpec((1, S, D), lambda h, i: (h, 0, 0)),
        ],
        out_specs=pl.BlockSpec((1, BQ, D), lambda h, i: (h, i, 0)),
        out_shape=jax.ShapeDtypeStruct((H, S, D), jnp.float32),
        compiler_params=pltpu.CompilerParams(
            dimension_semantics=("parallel", "parallel"),
        ),
    )(q, k, v)
    return out.reshape(B, H, S, D)
